# baseline (speedup 1.0000x reference)
import functools

import jax
import jax.numpy as jnp
from jax import lax
from jax.experimental import pallas as pl
from jax.experimental.pallas import tpu as pltpu
from jax.experimental.pallas import tpu_sc as plsc

_N_EMB = 32
_N_REGIONS = 4
_B, _T, _N = 512, 32, 256
_BT = _B * _T
_GRP = 16
_NBINS = _N_REGIONS * 64


def _sc_histogram(x_flat, off):
    info = plsc.get_sparse_core_info()
    nw = info.num_cores * info.num_subcores
    tok_per_w = _BT // nw
    groups = tok_per_w // _GRP

    mesh = plsc.VectorSubcoreMesh(core_axis_name="c", subcore_axis_name="s")

    @functools.partial(
        pl.kernel,
        mesh=mesh,
        compiler_params=pltpu.CompilerParams(needs_layout_passes=False),
        out_type=jax.ShapeDtypeStruct((_BT * _NBINS,), jnp.float32),
        scratch_types=[
            pltpu.VMEM((_GRP * _N,), jnp.int32),
            pltpu.VMEM((_GRP * _N,), jnp.int32),
            pltpu.VMEM((_GRP * _NBINS,), jnp.float32),
            pltpu.VMEM((_GRP * _NBINS,), jnp.float32),
            pltpu.VMEM((_N,), jnp.int32),
            pltpu.SemaphoreType.DMA,
            pltpu.SemaphoreType.DMA,
            pltpu.SemaphoreType.DMA,
            pltpu.SemaphoreType.DMA,
        ],
    )
    def k(x_hbm, off_hbm, counts_hbm, xblk0, xblk1, acc0, acc1, offv,
          sin0, sin1, sout0, sout1):
        wid = lax.axis_index("s") * info.num_cores + lax.axis_index("c")
        xblk = (xblk0, xblk1)
        acc = (acc0, acc1)
        sin = (sin0, sin1)
        sout = (sout0, sout1)
        pltpu.sync_copy(off_hbm, offv)
        ones = jnp.ones((16,), jnp.float32)
        zeros = jnp.zeros((16,), jnp.float32)
        offs = [offv[pl.ds(kk * 16, 16)] for kk in range(16)]

        def in_copy(g, b):
            tok0 = (wid * groups + g) * _GRP
            xoff = pl.multiple_of(tok0 * _N, 4096)
            return pltpu.make_async_copy(
                x_hbm.at[pl.ds(xoff, _GRP * _N)], xblk[b], sin[b])

        def out_copy(g, b):
            tok0 = (wid * groups + g) * _GRP
            obase = pl.multiple_of(tok0 * _NBINS, 4096)
            return pltpu.make_async_copy(
                acc[b], counts_hbm.at[pl.ds(obase, _GRP * _NBINS)], sout[b])

        def zero(b):
            def zbody(i, c):
                base = i * 256
                for kk in range(16):
                    acc[b][pl.ds(base + kk * 16, 16)] = zeros
                return c
            lax.fori_loop(0, _GRP * _NBINS // 256, zbody, 0)

        def hist(b):
            def hbody(tok, c):
                tbase = tok * _NBINS
                for kk in range(16):
                    xv = xblk[b][pl.ds(tok * _N + kk * 16, 16)]
                    plsc.addupdate_scatter(
                        acc[b], [xv + offs[kk] + tbase], ones)
                return c
            lax.fori_loop(0, _GRP, hbody, 0)

        in_copy(0, 0).start()
        in_copy(1, 1).start()
        zero(0)
        zero(1)

        def pair_body(gg, carry):
            for b in range(2):
                g = 2 * gg + b

                @pl.when(g >= 2)
                def _wait_prev_out():
                    out_copy(g - 2, b).wait()

                in_copy(g, b).wait()
                hist(b)
                out_copy(g, b).start()

                @pl.when(g + 2 < groups)
                def _prefetch():
                    in_copy(g + 2, b).start()
            return carry

        lax.fori_loop(0, groups // 2, pair_body, 0)
        out_copy(groups - 2, 0).wait()
        out_copy(groups - 1, 1).wait()

    return k(x_flat, off)


def _tc_assemble(counts, w, hemi_row):
    bb = 16
    n_out = 2 * _N_REGIONS * _N_EMB + 1

    def body(c_ref, w_ref, h_ref, o_ref):
        c = c_ref[...]
        prev = jnp.concatenate(
            [jnp.zeros((1, _T, _NBINS), jnp.float32), c[:-1]], axis=0)
        d = c - prev
        r = lax.dot_general(
            d, w_ref[...], (((2,), (0,)), ((), ())),
            preferred_element_type=jnp.float32)
        o_ref[:, :, 0:128] = r
        o_ref[:, :, 128:256] = jnp.zeros((bb, _T, 128), jnp.float32)
        o_ref[:, :, 256:257] = jnp.broadcast_to(
            h_ref[...][:, :, None], (bb, _T, 1))

    return pl.pallas_call(
        body,
        grid=(_B // bb,),
        in_specs=[
            pl.BlockSpec((bb, _T, _NBINS), lambda i: (i, 0, 0)),
            pl.BlockSpec((_NBINS, 128), lambda i: (0, 0)),
            pl.BlockSpec((1, _N_EMB), lambda i: (0, 0)),
        ],
        out_specs=pl.BlockSpec((bb, _T, n_out), lambda i: (i, 0, 0)),
        out_shape=jax.ShapeDtypeStruct((_B, _T, n_out), jnp.float32),
    )(counts, w, hemi_row)


def kernel(x, neuron_regions, is_left, table_0, table_1, table_2, table_3,
           hemi_table):
    nr = neuron_regions.astype(jnp.int32)
    x_flat = x.reshape(_BT * _N).astype(jnp.int32)
    off = nr * 64

    counts = _sc_histogram(x_flat, off).reshape(_B, _T, _NBINS)

    cnt = jnp.sum(
        (nr[:, None] == jnp.arange(_N_REGIONS, dtype=jnp.int32)[None, :])
        .astype(jnp.float32), axis=0)
    tables = jnp.stack([table_0, table_1, table_2, table_3])
    scaled = tables / cnt[:, None, None]
    w = (scaled[:, :, None, :]
         * jnp.eye(_N_REGIONS, dtype=jnp.float32)[:, None, :, None]
         ).reshape(_NBINS, _N_REGIONS * _N_EMB)
    hemi_row = hemi_table[is_left[0]][None, :]

    return _tc_assemble(counts, w, hemi_row)


# speedup vs baseline: 1.2614x; 1.2614x over previous
import functools

import jax
import jax.numpy as jnp
from jax import lax
from jax.experimental import pallas as pl
from jax.experimental.pallas import tpu as pltpu
from jax.experimental.pallas import tpu_sc as plsc

_N_EMB = 32
_N_REGIONS = 4
_B, _T, _N = 512, 32, 256
_BT = _B * _T
_GRP = 16
_NBINS = _N_REGIONS * 64


def _sc_histogram(x_flat, off):
    info = plsc.get_sparse_core_info()
    nw = info.num_cores * info.num_subcores
    tok_per_w = _BT // nw
    groups = tok_per_w // _GRP

    mesh = plsc.VectorSubcoreMesh(core_axis_name="c", subcore_axis_name="s")

    @functools.partial(
        pl.kernel,
        mesh=mesh,
        compiler_params=pltpu.CompilerParams(needs_layout_passes=False),
        out_type=jax.ShapeDtypeStruct((_BT * _NBINS,), jnp.float32),
        scratch_types=[
            pltpu.VMEM((_GRP * _N,), jnp.int32),
            pltpu.VMEM((_GRP * _N,), jnp.int32),
            pltpu.VMEM((_GRP * _NBINS,), jnp.float32),
            pltpu.VMEM((_GRP * _NBINS,), jnp.float32),
            pltpu.VMEM((_N,), jnp.int32),
            pltpu.SemaphoreType.DMA,
            pltpu.SemaphoreType.DMA,
            pltpu.SemaphoreType.DMA,
            pltpu.SemaphoreType.DMA,
        ],
    )
    def k(x_hbm, off_hbm, counts_hbm, xblk0, xblk1, acc0, acc1, offv,
          sin0, sin1, sout0, sout1):
        wid = lax.axis_index("s") * info.num_cores + lax.axis_index("c")
        xblk = (xblk0, xblk1)
        acc = (acc0, acc1)
        sin = (sin0, sin1)
        sout = (sout0, sout1)
        pltpu.sync_copy(off_hbm, offv)
        ones = jnp.ones((16,), jnp.float32)
        zeros = jnp.zeros((16,), jnp.float32)
        offs = [offv[pl.ds(kk * 16, 16)] for kk in range(16)]

        def in_copy(g, b):
            tok0 = (wid * groups + g) * _GRP
            xoff = pl.multiple_of(tok0 * _N, 4096)
            return pltpu.make_async_copy(
                x_hbm.at[pl.ds(xoff, _GRP * _N)], xblk[b], sin[b])

        def out_copy(g, b):
            tok0 = (wid * groups + g) * _GRP
            obase = pl.multiple_of(tok0 * _NBINS, 4096)
            return pltpu.make_async_copy(
                acc[b], counts_hbm.at[pl.ds(obase, _GRP * _NBINS)], sout[b])

        def zero(b):
            def zbody(i, c):
                base = i * 256
                for kk in range(16):
                    acc[b][pl.ds(base + kk * 16, 16)] = zeros
                return c
            lax.fori_loop(0, _GRP * _NBINS // 256, zbody, 0)

        def hist(b):
            @plsc.parallel_loop(0, _GRP, unroll=2)
            def hbody(tok):
                tbase = tok * _NBINS
                for kk in range(16):
                    xv = xblk[b][pl.ds(tok * _N + kk * 16, 16)]
                    plsc.addupdate_scatter(
                        acc[b], [xv + offs[kk] + tbase], ones)

        in_copy(0, 0).start()
        in_copy(1, 1).start()
        zero(0)
        zero(1)

        def pair_body(gg, carry):
            for b in range(2):
                g = 2 * gg + b

                @pl.when(g >= 2)
                def _wait_prev_out():
                    out_copy(g - 2, b).wait()

                in_copy(g, b).wait()
                hist(b)
                out_copy(g, b).start()

                @pl.when(g + 2 < groups)
                def _prefetch():
                    in_copy(g + 2, b).start()
            return carry

        lax.fori_loop(0, groups // 2, pair_body, 0)
        out_copy(groups - 2, 0).wait()
        out_copy(groups - 1, 1).wait()

    return k(x_flat, off)


def _tc_assemble(counts, w, hemi_row):
    bb = 16
    n_out = 2 * _N_REGIONS * _N_EMB + 1

    def body(c_ref, w_ref, h_ref, o_ref):
        c = c_ref[...]
        prev = jnp.concatenate(
            [jnp.zeros((1, _T, _NBINS), jnp.float32), c[:-1]], axis=0)
        d = c - prev
        r = lax.dot_general(
            d, w_ref[...], (((2,), (0,)), ((), ())),
            preferred_element_type=jnp.float32)
        o_ref[:, :, 0:128] = r
        o_ref[:, :, 128:256] = jnp.zeros((bb, _T, 128), jnp.float32)
        o_ref[:, :, 256:257] = jnp.broadcast_to(
            h_ref[...][:, :, None], (bb, _T, 1))

    return pl.pallas_call(
        body,
        grid=(_B // bb,),
        in_specs=[
            pl.BlockSpec((bb, _T, _NBINS), lambda i: (i, 0, 0)),
            pl.BlockSpec((_NBINS, 128), lambda i: (0, 0)),
            pl.BlockSpec((1, _N_EMB), lambda i: (0, 0)),
        ],
        out_specs=pl.BlockSpec((bb, _T, n_out), lambda i: (i, 0, 0)),
        out_shape=jax.ShapeDtypeStruct((_B, _T, n_out), jnp.float32),
    )(counts, w, hemi_row)


def kernel(x, neuron_regions, is_left, table_0, table_1, table_2, table_3,
           hemi_table):
    nr = neuron_regions.astype(jnp.int32)
    x_flat = x.reshape(_BT * _N).astype(jnp.int32)
    off = nr * 64

    counts = _sc_histogram(x_flat, off).reshape(_B, _T, _NBINS)

    cnt = jnp.sum(
        (nr[:, None] == jnp.arange(_N_REGIONS, dtype=jnp.int32)[None, :])
        .astype(jnp.float32), axis=0)
    tables = jnp.stack([table_0, table_1, table_2, table_3])
    scaled = tables / cnt[:, None, None]
    w = (scaled[:, :, None, :]
         * jnp.eye(_N_REGIONS, dtype=jnp.float32)[:, None, :, None]
         ).reshape(_NBINS, _N_REGIONS * _N_EMB)
    hemi_row = hemi_table[is_left[0]][None, :]

    return _tc_assemble(counts, w, hemi_row)
